# baseline (speedup 1.0000x reference)
import jax
import jax.numpy as jnp
from jax import lax
from jax.experimental import pallas as pl
from jax.experimental.pallas import tpu as pltpu
from jax.experimental.pallas import tpu_sc as plsc

NUM_TIMES = 288
NUM_WEEKDAYS = 7
TIME_DIM = 64
WEEKDAY_DIM = 64
MODEL_DIM = 512

NC = 2
NS = 16
NW = NC * NS

GROUPS = 16 * 12
GROUP = 512
TOKENS = GROUPS * GROUP
SC_GROUPS = 48
TC_GROUPS = GROUPS - SC_GROUPS
SC_TOKENS = SC_GROUPS * GROUP
ROWS_PER_W = SC_TOKENS // NW
CHUNK = 64
NBUF = 3
NCHUNK = ROWS_PER_W // CHUNK


def _tables_body(time_ref, wd_ref, tt_ref, wt_ref, wdt_ref, ww_ref,
                 c_ref, idx_ref, pt16_ref, pw_ref):
    p_time = jnp.dot(tt_ref[...], wt_ref[...],
                     preferred_element_type=jnp.float32)
    p_wd = jnp.dot(wdt_ref[...], ww_ref[...],
                   preferred_element_type=jnp.float32)
    c_ref[...] = p_time[:, None, :] + p_wd[None, :, :]
    idx_ref[...] = time_ref[...] * NUM_WEEKDAYS + wd_ref[...]
    pt16_ref[...] = p_time.astype(jnp.bfloat16)
    pw_ref[...] = p_wd


def _build_tables(time2d, wd2d, time_table, W_time, weekday_table, W_weekday):
    c3, idx, pt16, pw = pl.pallas_call(
        _tables_body,
        out_shape=[
            jax.ShapeDtypeStruct((NUM_TIMES, NUM_WEEKDAYS, MODEL_DIM),
                                 jnp.float32),
            jax.ShapeDtypeStruct(time2d.shape, jnp.int32),
            jax.ShapeDtypeStruct((NUM_TIMES, MODEL_DIM), jnp.bfloat16),
            jax.ShapeDtypeStruct((NUM_WEEKDAYS, MODEL_DIM), jnp.float32),
        ],
    )(time2d, wd2d, time_table, W_time, weekday_table, W_weekday)
    return (c3.reshape(NUM_TIMES * NUM_WEEKDAYS, MODEL_DIM),
            idx.reshape(-1), pt16, pw)


def _gather_body(c_hbm, idx_hbm, out_hbm, idx_v, bufs, gsems, wsems):
    wid = lax.axis_index("s") * NC + lax.axis_index("c")
    base = wid * ROWS_PER_W
    pltpu.sync_copy(idx_hbm.at[pl.ds(base, ROWS_PER_W)], idx_v)

    def gather(c, b):
        pltpu.async_copy(
            c_hbm.at[idx_v.at[pl.ds(c * CHUNK, CHUNK)]], bufs.at[b],
            gsems.at[b])

    def write(c, b):
        pltpu.async_copy(
            bufs.at[b], out_hbm.at[pl.ds(base + c * CHUNK, CHUNK)],
            wsems.at[b])

    def wait_gather(b):
        pltpu.make_async_copy(c_hbm.at[pl.ds(0, CHUNK)], bufs.at[b],
                              gsems.at[b]).wait()

    def wait_write(b):
        pltpu.make_async_copy(bufs.at[b], out_hbm.at[pl.ds(0, CHUNK)],
                              wsems.at[b]).wait()

    for b in range(NBUF):
        gather(b, b)

    def body(g):
        for b in range(NBUF):
            c = g + b
            wait_gather(b)
            write(c, b)
        for b in range(NBUF):
            nc = g + NBUF + b

            @pl.when(nc < NCHUNK)
            def _():
                wait_write(b)
                gather(nc, b)

    pl.loop(0, NCHUNK, step=NBUF)(body)
    for b in range(NBUF):
        wait_write(b)


def _sc_gather(combined, idx):
    mesh = plsc.VectorSubcoreMesh(core_axis_name="c", subcore_axis_name="s")
    run = pl.kernel(
        _gather_body,
        out_type=jax.ShapeDtypeStruct((TOKENS, MODEL_DIM), jnp.float32),
        mesh=mesh,
        scratch_types=[
            pltpu.VMEM((ROWS_PER_W,), jnp.int32),
            pltpu.VMEM((NBUF, CHUNK, MODEL_DIM), jnp.float32),
            pltpu.SemaphoreType.DMA((NBUF,)),
            pltpu.SemaphoreType.DMA((NBUF,)),
        ],
    )
    return run(combined, idx)


def _onehot_body(alias_ref, t_ref, wd_ref, pt_ref, pw_ref, out_ref):
    del alias_ref
    tvals = t_ref[0, 0]
    oh = (tvals[:, None] ==
          lax.broadcasted_iota(jnp.int32, (GROUP, NUM_TIMES), 1)
          ).astype(jnp.bfloat16)
    acc = jnp.dot(oh, pt_ref[...], preferred_element_type=jnp.float32)
    wd = wd_ref[0, 0, 0]
    wdoh = (lax.broadcasted_iota(jnp.int32, (NUM_WEEKDAYS, 1), 0) == wd
            ).astype(jnp.float32)
    row = jnp.sum(pw_ref[...] * wdoh, axis=0, keepdims=True)
    out_ref[0] = acc + row


def _tc_onehot(sc_out, time2d, wd2d, pt16, pw):
    out3 = sc_out.reshape(GROUPS, GROUP, MODEL_DIM)
    return pl.pallas_call(
        _onehot_body,
        grid=(TC_GROUPS,),
        in_specs=[
            pl.BlockSpec(memory_space=pltpu.MemorySpace.HBM),
            pl.BlockSpec((1, 1, GROUP), lambda i: (i + SC_GROUPS, 0, 0)),
            pl.BlockSpec((1, 1, 1), lambda i: (i + SC_GROUPS, 0, 0)),
            pl.BlockSpec((NUM_TIMES, MODEL_DIM), lambda i: (0, 0)),
            pl.BlockSpec((NUM_WEEKDAYS, MODEL_DIM), lambda i: (0, 0)),
        ],
        out_specs=pl.BlockSpec((1, GROUP, MODEL_DIM),
                               lambda i: (i + SC_GROUPS, 0, 0)),
        out_shape=jax.ShapeDtypeStruct((GROUPS, GROUP, MODEL_DIM),
                                       jnp.float32),
        input_output_aliases={0: 0},
    )(out3, time2d.reshape(GROUPS, 1, GROUP), wd2d.reshape(GROUPS, 1, 1),
      pt16, pw)


@jax.jit
def kernel(time, weekday, time_table, W_time, weekday_table, W_weekday):
    B, T, N = time.shape
    time2d = time.reshape(B * T, N).astype(jnp.int32)
    wd2d = weekday.reshape(B * T, 1).astype(jnp.int32)
    combined, idx, pt16, pw = _build_tables(
        time2d, wd2d, time_table, W_time, weekday_table, W_weekday)
    sc_out = _sc_gather(combined, idx)
    out = _tc_onehot(sc_out, time2d, wd2d, pt16, pw)
    return out.reshape(B, T, N, MODEL_DIM)


# speedup vs baseline: 1.3366x; 1.3366x over previous
import jax
import jax.numpy as jnp
from jax import lax
from jax.experimental import pallas as pl
from jax.experimental.pallas import tpu as pltpu
from jax.experimental.pallas import tpu_sc as plsc

NUM_TIMES = 288
NUM_WEEKDAYS = 7
TIME_DIM = 64
WEEKDAY_DIM = 64
MODEL_DIM = 512

NC = 2
NS = 16
NW = NC * NS

GROUPS = 16 * 12
GROUP = 512
TOKENS = GROUPS * GROUP
SC_GROUPS = 48
TC_GROUPS = GROUPS - SC_GROUPS
SC_TOKENS = SC_GROUPS * GROUP
ROWS_PER_W = SC_TOKENS // NW
CHUNK = 64
NBUF = 3
NCHUNK = ROWS_PER_W // CHUNK


def _tables_body(time_ref, wd_ref, tt_ref, wt_ref, wdt_ref, ww_ref,
                 c_ref, idx_ref, pt16_ref, pw_ref):
    p_time = jnp.dot(tt_ref[...], wt_ref[...],
                     preferred_element_type=jnp.float32)
    p_wd = jnp.dot(wdt_ref[...], ww_ref[...],
                   preferred_element_type=jnp.float32)
    c_ref[...] = p_time[:, None, :] + p_wd[None, :, :]
    idx_ref[...] = time_ref[...] * NUM_WEEKDAYS + wd_ref[...]
    pt16_ref[...] = p_time.astype(jnp.bfloat16)
    pw_ref[...] = p_wd


def _build_tables(time2d, wd2d, time_table, W_time, weekday_table, W_weekday):
    c3, idx, pt16, pw = pl.pallas_call(
        _tables_body,
        out_shape=[
            jax.ShapeDtypeStruct((NUM_TIMES, NUM_WEEKDAYS, MODEL_DIM),
                                 jnp.float32),
            jax.ShapeDtypeStruct(time2d.shape, jnp.int32),
            jax.ShapeDtypeStruct((NUM_TIMES, MODEL_DIM), jnp.bfloat16),
            jax.ShapeDtypeStruct((NUM_WEEKDAYS, MODEL_DIM), jnp.float32),
        ],
    )(time2d, wd2d, time_table, W_time, weekday_table, W_weekday)
    return (c3.reshape(NUM_TIMES * NUM_WEEKDAYS, MODEL_DIM),
            idx.reshape(-1), pt16, pw)


def _gather_body(c_hbm, idx_hbm, out_hbm, idx_v, bufs, gsems, wsems):
    wid = lax.axis_index("s") * NC + lax.axis_index("c")
    base = wid * ROWS_PER_W
    pltpu.sync_copy(idx_hbm.at[pl.ds(base, ROWS_PER_W)], idx_v)

    def gather(c, b):
        pltpu.async_copy(
            c_hbm.at[idx_v.at[pl.ds(c * CHUNK, CHUNK)]], bufs.at[b],
            gsems.at[b])

    def write(c, b):
        pltpu.async_copy(
            bufs.at[b], out_hbm.at[pl.ds(base + c * CHUNK, CHUNK)],
            wsems.at[b])

    def wait_gather(b):
        pltpu.make_async_copy(c_hbm.at[pl.ds(0, CHUNK)], bufs.at[b],
                              gsems.at[b]).wait()

    def wait_write(b):
        pltpu.make_async_copy(bufs.at[b], out_hbm.at[pl.ds(0, CHUNK)],
                              wsems.at[b]).wait()

    for b in range(NBUF):
        gather(b, b)

    def body(g):
        for b in range(NBUF):
            c = g + b
            wait_gather(b)
            write(c, b)
        for b in range(NBUF):
            nc = g + NBUF + b

            @pl.when(nc < NCHUNK)
            def _():
                wait_write(b)
                gather(nc, b)

    pl.loop(0, NCHUNK, step=NBUF)(body)
    for b in range(NBUF):
        wait_write(b)


def _sc_gather(combined, idx):
    mesh = plsc.VectorSubcoreMesh(core_axis_name="c", subcore_axis_name="s")
    run = pl.kernel(
        _gather_body,
        out_type=jax.ShapeDtypeStruct((TOKENS, MODEL_DIM), jnp.float32),
        mesh=mesh,
        scratch_types=[
            pltpu.VMEM((ROWS_PER_W,), jnp.int32),
            pltpu.VMEM((NBUF, CHUNK, MODEL_DIM), jnp.float32),
            pltpu.SemaphoreType.DMA((NBUF,)),
            pltpu.SemaphoreType.DMA((NBUF,)),
        ],
    )
    return run(combined, idx)


GB = 4
TC_STEPS = TC_GROUPS // GB


def _onehot_body(alias_ref, t_ref, wd_ref, pt_ref, pw_ref, out_ref):
    del alias_ref
    tvals = t_ref[0, 0]
    iota = lax.broadcasted_iota(jnp.int32, (GB * GROUP, NUM_TIMES), 1)
    oh = (tvals[:, None] == iota).astype(jnp.bfloat16)
    acc = jnp.dot(oh[:, :256], pt_ref[:256],
                  preferred_element_type=jnp.float32)
    acc = acc + jnp.dot(oh[:, 256:], pt_ref[256:],
                        preferred_element_type=jnp.float32)
    acc = acc.reshape(GB, GROUP, MODEL_DIM)
    wvals = wd_ref[0, 0]
    wdoh = (wvals[:, None] ==
            lax.broadcasted_iota(jnp.int32, (GB, NUM_WEEKDAYS), 1)
            ).astype(jnp.float32)
    rows = jnp.dot(wdoh, pw_ref[...],
                   preferred_element_type=jnp.float32)
    out_ref[0] = acc + rows[:, None, :]


def _tc_onehot(sc_out, time2d, wd2d, pt16, pw):
    out3 = sc_out.reshape(GROUPS // GB, GB, GROUP, MODEL_DIM)
    return pl.pallas_call(
        _onehot_body,
        grid=(TC_STEPS,),
        in_specs=[
            pl.BlockSpec(memory_space=pltpu.MemorySpace.HBM),
            pl.BlockSpec((1, 1, GB * GROUP),
                         lambda i: (i + SC_GROUPS // GB, 0, 0)),
            pl.BlockSpec((1, 1, GB), lambda i: (i + SC_GROUPS // GB, 0, 0)),
            pl.BlockSpec((NUM_TIMES, MODEL_DIM), lambda i: (0, 0)),
            pl.BlockSpec((NUM_WEEKDAYS, MODEL_DIM), lambda i: (0, 0)),
        ],
        out_specs=pl.BlockSpec((1, GB, GROUP, MODEL_DIM),
                               lambda i: (i + SC_GROUPS // GB, 0, 0, 0)),
        out_shape=jax.ShapeDtypeStruct((GROUPS // GB, GB, GROUP, MODEL_DIM),
                                       jnp.float32),
        input_output_aliases={0: 0},
    )(out3, time2d.reshape(GROUPS // GB, 1, GB * GROUP),
      wd2d.reshape(GROUPS // GB, 1, GB), pt16, pw)


@jax.jit
def kernel(time, weekday, time_table, W_time, weekday_table, W_weekday):
    B, T, N = time.shape
    time2d = time.reshape(B * T, N).astype(jnp.int32)
    wd2d = weekday.reshape(B * T, 1).astype(jnp.int32)
    combined, idx, pt16, pw = _build_tables(
        time2d, wd2d, time_table, W_time, weekday_table, W_weekday)
    sc_out = _sc_gather(combined, idx)
    out = _tc_onehot(sc_out, time2d, wd2d, pt16, pw)
    return out.reshape(B, T, N, MODEL_DIM)


# speedup vs baseline: 1.4020x; 1.0489x over previous
import jax
import jax.numpy as jnp
from jax import lax
from jax.experimental import pallas as pl
from jax.experimental.pallas import tpu as pltpu
from jax.experimental.pallas import tpu_sc as plsc

NUM_TIMES = 288
NUM_WEEKDAYS = 7
TIME_DIM = 64
WEEKDAY_DIM = 64
MODEL_DIM = 512

NC = 2
NS = 16
NW = NC * NS

GROUPS = 16 * 12
GROUP = 512
TOKENS = GROUPS * GROUP
SC_GROUPS = 48
TC_GROUPS = GROUPS - SC_GROUPS
SC_TOKENS = SC_GROUPS * GROUP
ROWS_PER_W = SC_TOKENS // NW
CHUNK = 64
NBUF = 3
NCHUNK = ROWS_PER_W // CHUNK


def _tables_body(time_ref, wd_ref, tt_ref, wt_ref, wdt_ref, ww_ref,
                 c_ref, idx_ref, pt16_ref, pw_ref):
    p_time = jnp.dot(tt_ref[...], wt_ref[...],
                     preferred_element_type=jnp.float32)
    p_wd = jnp.dot(wdt_ref[...], ww_ref[...],
                   preferred_element_type=jnp.float32)
    c_ref[...] = p_time[:, None, :] + p_wd[None, :, :]
    idx_ref[...] = time_ref[...] * NUM_WEEKDAYS + wd_ref[...]
    pt16_ref[...] = p_time.astype(jnp.bfloat16)
    pw_ref[...] = p_wd


def _build_tables(time2d, wd2d, time_table, W_time, weekday_table, W_weekday):
    c3, idx, pt16, pw = pl.pallas_call(
        _tables_body,
        out_shape=[
            jax.ShapeDtypeStruct((NUM_TIMES, NUM_WEEKDAYS, MODEL_DIM),
                                 jnp.float32),
            jax.ShapeDtypeStruct(time2d.shape, jnp.int32),
            jax.ShapeDtypeStruct((NUM_TIMES, MODEL_DIM), jnp.bfloat16),
            jax.ShapeDtypeStruct((NUM_WEEKDAYS, MODEL_DIM), jnp.float32),
        ],
    )(time2d, wd2d, time_table, W_time, weekday_table, W_weekday)
    return (c3.reshape(NUM_TIMES * NUM_WEEKDAYS, MODEL_DIM),
            idx.reshape(-1), pt16, pw)


def _gather_body(c_hbm, idx_hbm, out_hbm, idx_v, bufs, gsems, wsems):
    wid = lax.axis_index("s") * NC + lax.axis_index("c")
    base = wid * ROWS_PER_W
    pltpu.sync_copy(idx_hbm.at[pl.ds(base, ROWS_PER_W)], idx_v)

    def gather(c, b):
        pltpu.async_copy(
            c_hbm.at[idx_v.at[pl.ds(c * CHUNK, CHUNK)]], bufs.at[b],
            gsems.at[b])

    def write(c, b):
        pltpu.async_copy(
            bufs.at[b], out_hbm.at[pl.ds(base + c * CHUNK, CHUNK)],
            wsems.at[b])

    def wait_gather(b):
        pltpu.make_async_copy(c_hbm.at[pl.ds(0, CHUNK)], bufs.at[b],
                              gsems.at[b]).wait()

    def wait_write(b):
        pltpu.make_async_copy(bufs.at[b], out_hbm.at[pl.ds(0, CHUNK)],
                              wsems.at[b]).wait()

    for b in range(NBUF):
        gather(b, b)

    def body(g):
        for b in range(NBUF):
            c = g + b
            wait_gather(b)
            write(c, b)
        for b in range(NBUF):
            nc = g + NBUF + b

            @pl.when(nc < NCHUNK)
            def _():
                wait_write(b)
                gather(nc, b)

    pl.loop(0, NCHUNK, step=NBUF)(body)
    for b in range(NBUF):
        wait_write(b)


def _sc_gather(combined, idx):
    mesh = plsc.VectorSubcoreMesh(core_axis_name="c", subcore_axis_name="s")
    run = pl.kernel(
        _gather_body,
        out_type=jax.ShapeDtypeStruct((TOKENS, MODEL_DIM), jnp.float32),
        mesh=mesh,
        scratch_types=[
            pltpu.VMEM((ROWS_PER_W,), jnp.int32),
            pltpu.VMEM((NBUF, CHUNK, MODEL_DIM), jnp.float32),
            pltpu.SemaphoreType.DMA((NBUF,)),
            pltpu.SemaphoreType.DMA((NBUF,)),
        ],
    )
    return run(combined, idx)


GB = 8
TC_STEPS = TC_GROUPS // GB


def _onehot_body(alias_ref, t_ref, wd_ref, pt_ref, pw_ref, out_ref):
    del alias_ref
    tvals = t_ref[0, 0]
    iota = lax.broadcasted_iota(jnp.int32, (GB * GROUP, NUM_TIMES), 1)
    oh = (tvals[:, None] == iota).astype(jnp.bfloat16)
    acc = jnp.dot(oh[:, :256], pt_ref[:256],
                  preferred_element_type=jnp.float32)
    acc = acc + jnp.dot(oh[:, 256:], pt_ref[256:],
                        preferred_element_type=jnp.float32)
    acc = acc.reshape(GB, GROUP, MODEL_DIM)
    wvals = wd_ref[0, 0]
    wdoh = (wvals[:, None] ==
            lax.broadcasted_iota(jnp.int32, (GB, NUM_WEEKDAYS), 1)
            ).astype(jnp.float32)
    rows = jnp.dot(wdoh, pw_ref[...],
                   preferred_element_type=jnp.float32)
    out_ref[0] = acc + rows[:, None, :]


def _tc_onehot(sc_out, time2d, wd2d, pt16, pw):
    out3 = sc_out.reshape(GROUPS // GB, GB, GROUP, MODEL_DIM)
    return pl.pallas_call(
        _onehot_body,
        grid=(TC_STEPS,),
        in_specs=[
            pl.BlockSpec(memory_space=pltpu.MemorySpace.HBM),
            pl.BlockSpec((1, 1, GB * GROUP),
                         lambda i: (i + SC_GROUPS // GB, 0, 0)),
            pl.BlockSpec((1, 1, GB), lambda i: (i + SC_GROUPS // GB, 0, 0)),
            pl.BlockSpec((NUM_TIMES, MODEL_DIM), lambda i: (0, 0)),
            pl.BlockSpec((NUM_WEEKDAYS, MODEL_DIM), lambda i: (0, 0)),
        ],
        out_specs=pl.BlockSpec((1, GB, GROUP, MODEL_DIM),
                               lambda i: (i + SC_GROUPS // GB, 0, 0, 0)),
        out_shape=jax.ShapeDtypeStruct((GROUPS // GB, GB, GROUP, MODEL_DIM),
                                       jnp.float32),
        input_output_aliases={0: 0},
    )(out3, time2d.reshape(GROUPS // GB, 1, GB * GROUP),
      wd2d.reshape(GROUPS // GB, 1, GB), pt16, pw)


@jax.jit
def kernel(time, weekday, time_table, W_time, weekday_table, W_weekday):
    B, T, N = time.shape
    time2d = time.reshape(B * T, N).astype(jnp.int32)
    wd2d = weekday.reshape(B * T, 1).astype(jnp.int32)
    combined, idx, pt16, pw = _build_tables(
        time2d, wd2d, time_table, W_time, weekday_table, W_weekday)
    sc_out = _sc_gather(combined, idx)
    out = _tc_onehot(sc_out, time2d, wd2d, pt16, pw)
    return out.reshape(B, T, N, MODEL_DIM)


# speedup vs baseline: 1.4030x; 1.0007x over previous
import jax
import jax.numpy as jnp
from jax import lax
from jax.experimental import pallas as pl
from jax.experimental.pallas import tpu as pltpu
from jax.experimental.pallas import tpu_sc as plsc

NUM_TIMES = 288
NUM_WEEKDAYS = 7
TIME_DIM = 64
WEEKDAY_DIM = 64
MODEL_DIM = 512

NC = 2
NS = 16
NW = NC * NS

GROUPS = 16 * 12
GROUP = 512
TOKENS = GROUPS * GROUP
SC_GROUPS = 48
TC_GROUPS = GROUPS - SC_GROUPS
SC_TOKENS = SC_GROUPS * GROUP
ROWS_PER_W = SC_TOKENS // NW
CHUNK = 64
NBUF = 3
NCHUNK = ROWS_PER_W // CHUNK


def _tables_body(time_ref, wd_ref, tt_ref, wt_ref, wdt_ref, ww_ref,
                 c_ref, idx_ref, pt16_ref, pw_ref):
    p_time = jnp.dot(tt_ref[...], wt_ref[...],
                     preferred_element_type=jnp.float32)
    p_wd = jnp.dot(wdt_ref[...], ww_ref[...],
                   preferred_element_type=jnp.float32)
    c_ref[...] = p_time[:, None, :] + p_wd[None, :, :]
    idx_ref[...] = time_ref[...] * NUM_WEEKDAYS + wd_ref[...]
    pt16_ref[...] = p_time.astype(jnp.bfloat16)
    pw_ref[...] = p_wd


def _build_tables(time2d, wd2d, time_table, W_time, weekday_table, W_weekday):
    c3, idx, pt16, pw = pl.pallas_call(
        _tables_body,
        out_shape=[
            jax.ShapeDtypeStruct((NUM_TIMES, NUM_WEEKDAYS, MODEL_DIM),
                                 jnp.float32),
            jax.ShapeDtypeStruct(time2d.shape, jnp.int32),
            jax.ShapeDtypeStruct((NUM_TIMES, MODEL_DIM), jnp.bfloat16),
            jax.ShapeDtypeStruct((NUM_WEEKDAYS, MODEL_DIM), jnp.float32),
        ],
    )(time2d, wd2d, time_table, W_time, weekday_table, W_weekday)
    return (c3.reshape(NUM_TIMES * NUM_WEEKDAYS, MODEL_DIM),
            idx.reshape(-1), pt16, pw)


def _gather_body(c_hbm, idx_hbm, out_hbm, idx_v, bufs, gsems, wsems):
    wid = lax.axis_index("s") * NC + lax.axis_index("c")
    base = wid * ROWS_PER_W
    pltpu.sync_copy(idx_hbm.at[pl.ds(base, ROWS_PER_W)], idx_v)

    def gather(c, b):
        pltpu.async_copy(
            c_hbm.at[idx_v.at[pl.ds(c * CHUNK, CHUNK)]], bufs.at[b],
            gsems.at[b])

    def write(c, b):
        pltpu.async_copy(
            bufs.at[b], out_hbm.at[pl.ds(base + c * CHUNK, CHUNK)],
            wsems.at[b])

    def wait_gather(b):
        pltpu.make_async_copy(c_hbm.at[pl.ds(0, CHUNK)], bufs.at[b],
                              gsems.at[b]).wait()

    def wait_write(b):
        pltpu.make_async_copy(bufs.at[b], out_hbm.at[pl.ds(0, CHUNK)],
                              wsems.at[b]).wait()

    for b in range(NBUF):
        gather(b, b)

    def body(g):
        for b in range(NBUF):
            c = g + b
            wait_gather(b)
            write(c, b)
        for b in range(NBUF):
            nc = g + NBUF + b

            @pl.when(nc < NCHUNK)
            def _():
                wait_write(b)
                gather(nc, b)

    pl.loop(0, NCHUNK, step=NBUF)(body)
    for b in range(NBUF):
        wait_write(b)


def _sc_gather(combined, idx):
    mesh = plsc.VectorSubcoreMesh(core_axis_name="c", subcore_axis_name="s")
    run = pl.kernel(
        _gather_body,
        out_type=jax.ShapeDtypeStruct((TOKENS, MODEL_DIM), jnp.float32),
        mesh=mesh,
        scratch_types=[
            pltpu.VMEM((ROWS_PER_W,), jnp.int32),
            pltpu.VMEM((NBUF, CHUNK, MODEL_DIM), jnp.float32),
            pltpu.SemaphoreType.DMA((NBUF,)),
            pltpu.SemaphoreType.DMA((NBUF,)),
        ],
    )
    return run(combined, idx)


GB = 16
TC_STEPS = TC_GROUPS // GB


def _onehot_body(alias_ref, t_ref, wd_ref, pt_ref, pw_ref, out_ref):
    del alias_ref
    tvals = t_ref[0, 0]
    iota = lax.broadcasted_iota(jnp.int32, (GB * GROUP, NUM_TIMES), 1)
    oh = (tvals[:, None] == iota).astype(jnp.bfloat16)
    acc = jnp.dot(oh[:, :256], pt_ref[:256],
                  preferred_element_type=jnp.float32)
    acc = acc + jnp.dot(oh[:, 256:], pt_ref[256:],
                        preferred_element_type=jnp.float32)
    acc = acc.reshape(GB, GROUP, MODEL_DIM)
    wvals = wd_ref[0, 0]
    wdoh = (wvals[:, None] ==
            lax.broadcasted_iota(jnp.int32, (GB, NUM_WEEKDAYS), 1)
            ).astype(jnp.float32)
    rows = jnp.dot(wdoh, pw_ref[...],
                   preferred_element_type=jnp.float32)
    out_ref[0] = acc + rows[:, None, :]


def _tc_onehot(sc_out, time2d, wd2d, pt16, pw):
    out3 = sc_out.reshape(GROUPS // GB, GB, GROUP, MODEL_DIM)
    return pl.pallas_call(
        _onehot_body,
        grid=(TC_STEPS,),
        in_specs=[
            pl.BlockSpec(memory_space=pltpu.MemorySpace.HBM),
            pl.BlockSpec((1, 1, GB * GROUP),
                         lambda i: (i + SC_GROUPS // GB, 0, 0)),
            pl.BlockSpec((1, 1, GB), lambda i: (i + SC_GROUPS // GB, 0, 0)),
            pl.BlockSpec((NUM_TIMES, MODEL_DIM), lambda i: (0, 0)),
            pl.BlockSpec((NUM_WEEKDAYS, MODEL_DIM), lambda i: (0, 0)),
        ],
        out_specs=pl.BlockSpec((1, GB, GROUP, MODEL_DIM),
                               lambda i: (i + SC_GROUPS // GB, 0, 0, 0)),
        out_shape=jax.ShapeDtypeStruct((GROUPS // GB, GB, GROUP, MODEL_DIM),
                                       jnp.float32),
        input_output_aliases={0: 0},
    )(out3, time2d.reshape(GROUPS // GB, 1, GB * GROUP),
      wd2d.reshape(GROUPS // GB, 1, GB), pt16, pw)


@jax.jit
def kernel(time, weekday, time_table, W_time, weekday_table, W_weekday):
    B, T, N = time.shape
    time2d = time.reshape(B * T, N).astype(jnp.int32)
    wd2d = weekday.reshape(B * T, 1).astype(jnp.int32)
    combined, idx, pt16, pw = _build_tables(
        time2d, wd2d, time_table, W_time, weekday_table, W_weekday)
    sc_out = _sc_gather(combined, idx)
    out = _tc_onehot(sc_out, time2d, wd2d, pt16, pw)
    return out.reshape(B, T, N, MODEL_DIM)


# speedup vs baseline: 1.4801x; 1.0549x over previous
import jax
import jax.numpy as jnp
from jax import lax
from jax.experimental import pallas as pl
from jax.experimental.pallas import tpu as pltpu
from jax.experimental.pallas import tpu_sc as plsc

NUM_TIMES = 288
NUM_WEEKDAYS = 7
TIME_DIM = 64
WEEKDAY_DIM = 64
MODEL_DIM = 512

NC = 2
NS = 16
NW = NC * NS

GROUPS = 16 * 12
GROUP = 512
TOKENS = GROUPS * GROUP
SC_GROUPS = 32
TC_GROUPS = GROUPS - SC_GROUPS
SC_TOKENS = SC_GROUPS * GROUP
ROWS_PER_W = SC_TOKENS // NW
CHUNK = 64
NBUF = 2
NCHUNK = ROWS_PER_W // CHUNK


def _tables_body(time_ref, wd_ref, tt_ref, wt_ref, wdt_ref, ww_ref,
                 c_ref, idx_ref, pt16_ref, pw_ref):
    p_time = jnp.dot(tt_ref[...], wt_ref[...],
                     preferred_element_type=jnp.float32)
    p_wd = jnp.dot(wdt_ref[...], ww_ref[...],
                   preferred_element_type=jnp.float32)
    c_ref[...] = p_time[:, None, :] + p_wd[None, :, :]
    idx_ref[...] = time_ref[...] * NUM_WEEKDAYS + wd_ref[...]
    pt16_ref[...] = p_time.astype(jnp.bfloat16)
    pw_ref[...] = p_wd


def _build_tables(time2d, wd2d, time_table, W_time, weekday_table, W_weekday):
    c3, idx, pt16, pw = pl.pallas_call(
        _tables_body,
        out_shape=[
            jax.ShapeDtypeStruct((NUM_TIMES, NUM_WEEKDAYS, MODEL_DIM),
                                 jnp.float32),
            jax.ShapeDtypeStruct(time2d.shape, jnp.int32),
            jax.ShapeDtypeStruct((NUM_TIMES, MODEL_DIM), jnp.bfloat16),
            jax.ShapeDtypeStruct((NUM_WEEKDAYS, MODEL_DIM), jnp.float32),
        ],
    )(time2d, wd2d, time_table, W_time, weekday_table, W_weekday)
    return (c3.reshape(NUM_TIMES * NUM_WEEKDAYS, MODEL_DIM),
            idx.reshape(-1), pt16, pw)


def _gather_body(c_hbm, idx_hbm, out_hbm, idx_v, bufs, gsems, wsems):
    wid = lax.axis_index("s") * NC + lax.axis_index("c")
    base = wid * ROWS_PER_W
    pltpu.sync_copy(idx_hbm.at[pl.ds(base, ROWS_PER_W)], idx_v)

    def gather(c, b):
        pltpu.async_copy(
            c_hbm.at[idx_v.at[pl.ds(c * CHUNK, CHUNK)]], bufs.at[b],
            gsems.at[b])

    def write(c, b):
        pltpu.async_copy(
            bufs.at[b], out_hbm.at[pl.ds(base + c * CHUNK, CHUNK)],
            wsems.at[b])

    def wait_gather(b):
        pltpu.make_async_copy(c_hbm.at[pl.ds(0, CHUNK)], bufs.at[b],
                              gsems.at[b]).wait()

    def wait_write(b):
        pltpu.make_async_copy(bufs.at[b], out_hbm.at[pl.ds(0, CHUNK)],
                              wsems.at[b]).wait()

    for b in range(NBUF):
        gather(b, b)

    def body(g):
        for b in range(NBUF):
            c = g + b
            wait_gather(b)
            write(c, b)
        for b in range(NBUF):
            nc = g + NBUF + b

            @pl.when(nc < NCHUNK)
            def _():
                wait_write(b)
                gather(nc, b)

    pl.loop(0, NCHUNK, step=NBUF)(body)
    for b in range(NBUF):
        wait_write(b)


def _sc_gather(combined, idx):
    mesh = plsc.VectorSubcoreMesh(core_axis_name="c", subcore_axis_name="s")
    run = pl.kernel(
        _gather_body,
        out_type=jax.ShapeDtypeStruct((TOKENS, MODEL_DIM), jnp.float32),
        mesh=mesh,
        scratch_types=[
            pltpu.VMEM((ROWS_PER_W,), jnp.int32),
            pltpu.VMEM((NBUF, CHUNK, MODEL_DIM), jnp.float32),
            pltpu.SemaphoreType.DMA((NBUF,)),
            pltpu.SemaphoreType.DMA((NBUF,)),
        ],
    )
    return run(combined, idx)


GB = 8
TC_STEPS = TC_GROUPS // GB


def _onehot_body(alias_ref, t_ref, wd_ref, pt_ref, pw_ref, out_ref):
    del alias_ref
    tvals = t_ref[0, 0]
    iota = lax.broadcasted_iota(jnp.int32, (GB * GROUP, NUM_TIMES), 1)
    oh = (tvals[:, None] == iota).astype(jnp.bfloat16)
    acc = jnp.dot(oh[:, :256], pt_ref[:256],
                  preferred_element_type=jnp.float32)
    acc = acc + jnp.dot(oh[:, 256:], pt_ref[256:],
                        preferred_element_type=jnp.float32)
    acc = acc.reshape(GB, GROUP, MODEL_DIM)
    wvals = wd_ref[0, 0]
    wdoh = (wvals[:, None] ==
            lax.broadcasted_iota(jnp.int32, (GB, NUM_WEEKDAYS), 1)
            ).astype(jnp.float32)
    rows = jnp.dot(wdoh, pw_ref[...],
                   preferred_element_type=jnp.float32)
    out_ref[0] = acc + rows[:, None, :]


def _tc_onehot(sc_out, time2d, wd2d, pt16, pw):
    out3 = sc_out.reshape(GROUPS // GB, GB, GROUP, MODEL_DIM)
    return pl.pallas_call(
        _onehot_body,
        grid=(TC_STEPS,),
        in_specs=[
            pl.BlockSpec(memory_space=pltpu.MemorySpace.HBM),
            pl.BlockSpec((1, 1, GB * GROUP),
                         lambda i: (i + SC_GROUPS // GB, 0, 0)),
            pl.BlockSpec((1, 1, GB), lambda i: (i + SC_GROUPS // GB, 0, 0)),
            pl.BlockSpec((NUM_TIMES, MODEL_DIM), lambda i: (0, 0)),
            pl.BlockSpec((NUM_WEEKDAYS, MODEL_DIM), lambda i: (0, 0)),
        ],
        out_specs=pl.BlockSpec((1, GB, GROUP, MODEL_DIM),
                               lambda i: (i + SC_GROUPS // GB, 0, 0, 0)),
        out_shape=jax.ShapeDtypeStruct((GROUPS // GB, GB, GROUP, MODEL_DIM),
                                       jnp.float32),
        input_output_aliases={0: 0},
    )(out3, time2d.reshape(GROUPS // GB, 1, GB * GROUP),
      wd2d.reshape(GROUPS // GB, 1, GB), pt16, pw)


@jax.jit
def kernel(time, weekday, time_table, W_time, weekday_table, W_weekday):
    B, T, N = time.shape
    time2d = time.reshape(B * T, N).astype(jnp.int32)
    wd2d = weekday.reshape(B * T, 1).astype(jnp.int32)
    combined, idx, pt16, pw = _build_tables(
        time2d, wd2d, time_table, W_time, weekday_table, W_weekday)
    sc_out = _sc_gather(combined, idx)
    out = _tc_onehot(sc_out, time2d, wd2d, pt16, pw)
    return out.reshape(B, T, N, MODEL_DIM)
